# Initial kernel scaffold; baseline (speedup 1.0000x reference)
import jax
import jax.numpy as jnp
from jax.experimental import pallas as pl


def kernel(fea, keypoints):
    raise NotImplementedError("write your pallas kernel here")


# speedup vs baseline: 29.6343x; 29.6343x over previous
import jax
import jax.numpy as jnp
from jax.experimental import pallas as pl
from jax.experimental.pallas import tpu as pltpu

_PH, _PW = 2, 2
_XY = 16.0
_G = 16
_CBLK = 8
_NKP = 5


def _roi_kernel(kp_ref, fea_ref, out_ref):
    i = pl.program_id(0)
    fea = fea_ref[0]
    H = 128.0
    W = 128.0
    L = 1024

    ls = jax.lax.broadcasted_iota(jnp.int32, (L, 8), 0).astype(jnp.float32)
    dd = jax.lax.broadcasted_iota(jnp.int32, (L, 8), 1).astype(jnp.float32)
    S = jnp.where(ls - 8.0 * jnp.floor(ls * 0.125) == dd, 1.0, 0.0)

    ay_rows = []
    ax_rows = []
    for j in range(_NKP):
        kpx = kp_ref[i, j, 0] * W
        kpy = kp_ref[i, j, 1] * H
        xmin = jnp.clip(kpx - _XY, 0.0, W - 1.0)
        xmax = jnp.clip(kpx + _XY, 0.0, W - 1.0)
        ymin = jnp.clip(kpy - _XY, 0.0, H - 1.0)
        ymax = jnp.clip(kpy + _XY, 0.0, H - 1.0)
        roi_w = jnp.maximum(xmax - xmin, 1.0)
        roi_h = jnp.maximum(ymax - ymin, 1.0)
        bh = roi_h / _PH
        bw = roi_w / _PW
        gh = jnp.ceil(roi_h / _PH)
        gw = jnp.ceil(roi_w / _PW)
        scale = 1.0 / (gh * gw)

        pyi = jax.lax.broadcasted_iota(jnp.int32, (_PH, _G, 128), 0).astype(jnp.float32)
        kyi = jax.lax.broadcasted_iota(jnp.int32, (_PH, _G, 128), 1).astype(jnp.float32)
        yi = jax.lax.broadcasted_iota(jnp.int32, (_PH, _G, 128), 2).astype(jnp.float32)
        ys = ymin + pyi * bh + (kyi + 0.5) * (bh / gh)
        hat_y = jnp.maximum(1.0 - jnp.abs(yi - ys), 0.0)
        ay = jnp.where(kyi < gh, hat_y, 0.0).sum(axis=1) * scale
        ay_rows.append(ay)

        pxi = jax.lax.broadcasted_iota(jnp.int32, (_PW, _G, L), 0).astype(jnp.float32)
        kxi = jax.lax.broadcasted_iota(jnp.int32, (_PW, _G, L), 1).astype(jnp.float32)
        li = jax.lax.broadcasted_iota(jnp.int32, (_PW, _G, L), 2).astype(jnp.float32)
        col = jnp.floor(li * 0.125)
        xs = xmin + pxi * bw + (kxi + 0.5) * (bw / gw)
        hat_x = jnp.maximum(1.0 - jnp.abs(col - xs), 0.0)
        ax = jnp.where(kxi < gw, hat_x, 0.0).sum(axis=1)
        ax_rows.append(ax)

    ay_all = jnp.concatenate(ay_rows, axis=0)

    ts = []
    for c in range(_CBLK):
        ts.append(jax.lax.dot_general(
            ay_all, fea[c], (((1,), (0,)), ((), ())),
            preferred_element_type=jnp.float32))
    t = jnp.stack(ts, axis=0)

    ps = []
    for j in range(_NKP):
        axj = ax_rows[j]
        for px in range(_PW):
            for py in range(_PH):
                ps.append(t[:, j * _PH + py, :] * axj[px][None, :])
    p = jnp.concatenate(ps, axis=0)
    o = jax.lax.dot_general(p, S, (((1,), (0,)), ((), ())),
                            preferred_element_type=jnp.float32)
    out_ref[...] = o.reshape(1, _NKP, _PW, _PH, _CBLK, 8)


@jax.jit
def kernel(fea, keypoints):
    with jax.enable_x64(False):
        return _run(fea, keypoints)


def _run(fea, keypoints):
    b, c, h, w, depth = fea.shape
    fea2 = fea.reshape(b, c, h, w * depth)
    kp = keypoints.astype(jnp.float32)
    out = pl.pallas_call(
        _roi_kernel,
        grid=(b, c // _CBLK),
        in_specs=[
            pl.BlockSpec(memory_space=pltpu.SMEM),
            pl.BlockSpec((1, _CBLK, h, w * depth), lambda i, cb: (i, cb, 0, 0)),
        ],
        out_specs=pl.BlockSpec((1, _NKP, _PW, _PH, _CBLK, 8),
                               lambda i, cb: (i, 0, 0, 0, cb, 0)),
        out_shape=jax.ShapeDtypeStruct((b, _NKP, _PW, _PH, c, depth),
                                       jnp.float32),
        compiler_params=pltpu.CompilerParams(
            dimension_semantics=("parallel", "parallel")),
    )(kp, fea2)
    return out.transpose(0, 1, 4, 3, 2, 5)


# speedup vs baseline: 68.3151x; 2.3053x over previous
import jax
import jax.numpy as jnp
from jax.experimental import pallas as pl
from jax.experimental.pallas import tpu as pltpu

_PH, _PW = 2, 2
_XY = 16.0
_G = 16
_CBLK = 8
_NKP = 5


def _roi_kernel(kp_ref, fea_ref, out_ref):
    i = pl.program_id(0)
    H = 128.0
    W = 128.0
    D = 8
    F = fea_ref[0].reshape(_CBLK * 128 * D, 128)

    ay_rows = []
    axt_cols = []
    for j in range(_NKP):
        kpx = kp_ref[i, j, 0] * W
        kpy = kp_ref[i, j, 1] * H
        xmin = jnp.clip(kpx - _XY, 0.0, W - 1.0)
        xmax = jnp.clip(kpx + _XY, 0.0, W - 1.0)
        ymin = jnp.clip(kpy - _XY, 0.0, H - 1.0)
        ymax = jnp.clip(kpy + _XY, 0.0, H - 1.0)
        roi_w = jnp.maximum(xmax - xmin, 1.0)
        roi_h = jnp.maximum(ymax - ymin, 1.0)
        bh = roi_h / _PH
        bw = roi_w / _PW
        gh = jnp.ceil(roi_h / _PH)
        gw = jnp.ceil(roi_w / _PW)
        scale = 1.0 / (gh * gw)

        pyi = jax.lax.broadcasted_iota(jnp.int32, (_PH, _G, 128), 0).astype(jnp.float32)
        kyi = jax.lax.broadcasted_iota(jnp.int32, (_PH, _G, 128), 1).astype(jnp.float32)
        yi = jax.lax.broadcasted_iota(jnp.int32, (_PH, _G, 128), 2).astype(jnp.float32)
        ys = ymin + pyi * bh + (kyi + 0.5) * (bh / gh)
        hat_y = jnp.maximum(1.0 - jnp.abs(yi - ys), 0.0)
        ay = jnp.where(kyi < gh, hat_y, 0.0).sum(axis=1) * scale
        ay_rows.append(ay)

        kxi = jax.lax.broadcasted_iota(jnp.int32, (_G, 128, _PW), 0).astype(jnp.float32)
        xi = jax.lax.broadcasted_iota(jnp.int32, (_G, 128, _PW), 1).astype(jnp.float32)
        pxi = jax.lax.broadcasted_iota(jnp.int32, (_G, 128, _PW), 2).astype(jnp.float32)
        xs = xmin + pxi * bw + (kxi + 0.5) * (bw / gw)
        hat_x = jnp.maximum(1.0 - jnp.abs(xi - xs), 0.0)
        axt = jnp.where(kxi < gw, hat_x, 0.0).sum(axis=0)
        axt_cols.append(axt)

    ay_all = jnp.concatenate(ay_rows, axis=0)
    axt_all = jnp.concatenate(axt_cols, axis=1)

    g = jax.lax.dot_general(F, axt_all, (((1,), (0,)), ((), ())),
                            preferred_element_type=jnp.float32)
    g4 = g.reshape(_CBLK, 128, D, 2 * _NKP)
    gt = jnp.transpose(g4, (1, 0, 2, 3)).reshape(128, _CBLK * D * 2 * _NKP)

    r = jax.lax.dot_general(ay_all, gt, (((1,), (0,)), ((), ())),
                            preferred_element_type=jnp.float32)
    r4 = r.reshape(2 * _NKP, _CBLK, D, 2 * _NKP)

    parts = []
    for j in range(_NKP):
        for px in range(_PW):
            for py in range(_PH):
                parts.append(r4[j * _PH + py, :, :, j * _PW + px])
    o = jnp.stack(parts, axis=0)
    out_ref[...] = o.reshape(1, _NKP, _PW, _PH, _CBLK, D)


@jax.jit
def kernel(fea, keypoints):
    with jax.enable_x64(False):
        return _run(fea, keypoints)


def _run(fea, keypoints):
    b, c, h, w, depth = fea.shape
    fea_t = fea.transpose(0, 1, 2, 4, 3)
    kp = keypoints.astype(jnp.float32)
    out = pl.pallas_call(
        _roi_kernel,
        grid=(b, c // _CBLK),
        in_specs=[
            pl.BlockSpec(memory_space=pltpu.SMEM),
            pl.BlockSpec((1, _CBLK, h, depth, w),
                         lambda i, cb: (i, cb, 0, 0, 0)),
        ],
        out_specs=pl.BlockSpec((1, _NKP, _PW, _PH, _CBLK, depth),
                               lambda i, cb: (i, 0, 0, 0, cb, 0)),
        out_shape=jax.ShapeDtypeStruct((b, _NKP, _PW, _PH, c, depth),
                                       jnp.float32),
        compiler_params=pltpu.CompilerParams(
            dimension_semantics=("parallel", "parallel")),
    )(kp, fea_t)
    return out.transpose(0, 1, 4, 3, 2, 5)
